# Initial kernel scaffold; baseline (speedup 1.0000x reference)
import jax
import jax.numpy as jnp
from jax.experimental import pallas as pl

B, N, D, K = 64, 1024, 128, 256
NUM_SQUARINGS = 7


def _vlad_kernel(cent_ref, c2_ref, descs_ref, out_ref):
    cent = cent_ref[...]
    c2 = c2_ref[...]
    d = descs_ref[0]

    scores = jax.lax.dot_general(
        d, cent, (((1,), (1,)), ((), ())),
        preferred_element_type=jnp.float32)
    dist = c2 - 2.0 * scores
    clusters = jnp.argmin(dist, axis=-1)

    k_iota = jax.lax.broadcasted_iota(jnp.int32, (N, K), 1)
    onehot = (clusters[:, None] == k_iota).astype(jnp.float32)
    desc_sums = jax.lax.dot_general(
        onehot, d, (((0,), (0,)), ((), ())),
        preferred_element_type=jnp.float32)
    pops = jnp.sum(onehot, axis=0)

    r = cent * pops[:, None] - desc_sums

    g = jax.lax.dot_general(
        r, r, (((0,), (0,)), ((), ())),
        preferred_element_type=jnp.float32)
    eye = (jax.lax.broadcasted_iota(jnp.int32, (D, D), 0)
           == jax.lax.broadcasted_iota(jnp.int32, (D, D), 1))
    tr = jnp.sum(jnp.where(eye, g, 0.0))
    h = (g / tr).astype(jnp.bfloat16)
    for _ in range(NUM_SQUARINGS):
        h2 = jax.lax.dot_general(
            h, h, (((1,), (0,)), ((), ())),
            preferred_element_type=jnp.float32)
        tr2 = jnp.sum(jnp.where(eye, h2, 0.0))
        h = (h2 / tr2).astype(jnp.bfloat16)
    y = jnp.sum(h.astype(jnp.float32), axis=1)
    z = jax.lax.dot_general(
        g, y[:, None], (((1,), (0,)), ((), ())),
        preferred_element_type=jnp.float32)[:, 0]
    lam = jnp.sum(y * z) / jnp.sum(y * y)
    inv_norm = jax.lax.rsqrt(lam)

    out_ref[0] = r * inv_norm


@jax.jit
def kernel(descs, centroids_sum, populations):
    centroids = centroids_sum / populations[:, None]
    c2 = jnp.sum(centroids * centroids, axis=-1)[None, :]
    return pl.pallas_call(
        _vlad_kernel,
        grid=(B,),
        in_specs=[
            pl.BlockSpec((K, D), lambda b: (0, 0)),
            pl.BlockSpec((1, K), lambda b: (0, 0)),
            pl.BlockSpec((1, N, D), lambda b: (b, 0, 0)),
        ],
        out_specs=pl.BlockSpec((1, K, D), lambda b: (b, 0, 0)),
        out_shape=jax.ShapeDtypeStruct((B, K, D), jnp.float32),
    )(centroids, c2, descs)


# speedup vs baseline: 80.4849x; 80.4849x over previous
import jax
import jax.numpy as jnp
from jax.experimental import pallas as pl

B, N, D, K = 64, 1024, 128, 256
NUM_SQUARINGS = 7


def _vlad_kernel(cent_ref, c2_ref, descs_ref, out_ref):
    cent = cent_ref[...]
    c2 = c2_ref[...]
    d = descs_ref[0]

    scores = jax.lax.dot_general(
        d, cent, (((1,), (1,)), ((), ())),
        preferred_element_type=jnp.float32,
        precision=jax.lax.Precision.HIGHEST)
    dist = c2 - 2.0 * scores
    clusters = jnp.argmin(dist, axis=-1)

    k_iota = jax.lax.broadcasted_iota(jnp.int32, (N, K), 1)
    onehot = (clusters[:, None] == k_iota).astype(jnp.float32)
    desc_sums = jax.lax.dot_general(
        onehot, d, (((0,), (0,)), ((), ())),
        preferred_element_type=jnp.float32,
        precision=jax.lax.Precision.HIGHEST)
    pops = jnp.sum(onehot, axis=0)

    r = cent * pops[:, None] - desc_sums

    g = jax.lax.dot_general(
        r, r, (((0,), (0,)), ((), ())),
        preferred_element_type=jnp.float32,
        precision=jax.lax.Precision.HIGHEST)
    eye = (jax.lax.broadcasted_iota(jnp.int32, (D, D), 0)
           == jax.lax.broadcasted_iota(jnp.int32, (D, D), 1))
    tr = jnp.sum(jnp.where(eye, g, 0.0))
    h = (g / tr).astype(jnp.bfloat16)
    for _ in range(NUM_SQUARINGS):
        h2 = jax.lax.dot_general(
            h, h, (((1,), (0,)), ((), ())),
            preferred_element_type=jnp.float32)
        tr2 = jnp.sum(jnp.where(eye, h2, 0.0))
        h = (h2 / tr2).astype(jnp.bfloat16)
    y = jnp.sum(h.astype(jnp.float32), axis=1)
    z = jax.lax.dot_general(
        g, y[:, None], (((1,), (0,)), ((), ())),
        preferred_element_type=jnp.float32,
        precision=jax.lax.Precision.HIGHEST)[:, 0]
    lam = jnp.sum(y * z) / jnp.sum(y * y)
    inv_norm = jax.lax.rsqrt(lam)

    out_ref[0] = r * inv_norm


@jax.jit
def kernel(descs, centroids_sum, populations):
    centroids = centroids_sum / populations[:, None]
    c2 = jnp.sum(centroids * centroids, axis=-1)[None, :]
    return pl.pallas_call(
        _vlad_kernel,
        grid=(B,),
        in_specs=[
            pl.BlockSpec((K, D), lambda b: (0, 0)),
            pl.BlockSpec((1, K), lambda b: (0, 0)),
            pl.BlockSpec((1, N, D), lambda b: (b, 0, 0)),
        ],
        out_specs=pl.BlockSpec((1, K, D), lambda b: (b, 0, 0)),
        out_shape=jax.ShapeDtypeStruct((B, K, D), jnp.float32),
    )(centroids, c2, descs)
